# baseline (speedup 1.0000x reference)
import functools

import jax
import jax.numpy as jnp
import numpy as np
from jax import lax
from jax.experimental import pallas as pl
from jax.experimental.pallas import tpu as pltpu
from jax.experimental.pallas import tpu_sc as plsc

_N_CODES = 8192
_D = 256
_N_TOK = 8192
_TN = 256
_GRID = _N_TOK // _TN

_COMMIT = 0.25


_WIN = 2048
_BIG = np.int32(1 << 30)


def _round_bf16(x):
    u = lax.bitcast_convert_type(x, jnp.uint32)
    r = (u + np.uint32(0x7FFF) + ((u >> 16) & np.uint32(1))) \
        & np.uint32(0xFFFF0000)
    return lax.bitcast_convert_type(r, jnp.float32)


def _argmin_body(prec, z_ref, et_ref, idx_ref, loss_ref, esq_ref, etb_ref):
    i = pl.program_id(0)

    @pl.when(i == 0)
    def _init():
        et = et_ref[...]
        esq_ref[...] = jnp.sum(et * et, axis=0, keepdims=True)
        etb_ref[...] = et.astype(jnp.bfloat16)
        loss_ref[...] = jnp.zeros_like(loss_ref)

    zt = z_ref[...]
    zsq = jnp.sum(zt * zt, axis=1, keepdims=True)
    zb2 = (zt + zt).astype(jnp.bfloat16)
    accv = acci = accf = None
    for w in range(_N_CODES // _WIN):
        mm2 = lax.dot_general(zb2, etb_ref[:, w * _WIN:(w + 1) * _WIN],
                              (((1,), (0,)), ((), ())),
                              preferred_element_type=jnp.float32)
        dw = (zsq + esq_ref[:, w * _WIN:(w + 1) * _WIN]) - mm2
        wmin = jnp.min(dw, axis=1, keepdims=True)
        ii = lax.broadcasted_iota(jnp.int32, dw.shape, 1) + np.int32(w * _WIN)
        warg = jnp.min(jnp.where(dw == wmin, ii, _BIG),
                       axis=1, keepdims=True)
        if w == 0:
            accv, acci, accf = _round_bf16(wmin), warg, wmin
        else:
            take_acc = (accv < wmin) | ((accv == wmin) & (acci < warg))
            accv = _round_bf16(jnp.where(take_acc, accv, wmin))
            acci = jnp.where(take_acc, acci, warg)
            accf = jnp.where(take_acc, accf, wmin)
    idx_ref[0] = acci
    loss_ref[...] = loss_ref[...] + jnp.sum(accf, axis=(0, 1), keepdims=True)


def _vq_argmin(z_flat, emb_t, prec=lax.Precision.DEFAULT):
    idx3, loss = pl.pallas_call(
        functools.partial(_argmin_body, prec),
        grid=(_GRID,),
        in_specs=[pl.BlockSpec((_TN, _D), lambda i: (i, 0)),
                  pl.BlockSpec((_D, _N_CODES), lambda i: (0, 0))],
        out_specs=[pl.BlockSpec((1, _TN, 1), lambda i: (i, 0, 0)),
                   pl.BlockSpec((1, 1), lambda i: (0, 0))],
        out_shape=[jax.ShapeDtypeStruct((_GRID, _TN, 1), jnp.int32),
                   jax.ShapeDtypeStruct((1, 1), jnp.float32)],
        scratch_shapes=[pltpu.VMEM((1, _N_CODES), jnp.float32),
                        pltpu.VMEM((_D, _N_CODES), jnp.bfloat16)],
    )(z_flat, emb_t)
    return idx3.reshape(-1), loss[0, 0]


_NW = 32
_BPW = _N_TOK // _NW
_CH = 128
_NCHUNK = _BPW // _CH


def _gather_body(nc, table_hbm, idx_hbm, out_hbm, idx_v, rows_v, sem):
    wid = lax.axis_index("s") * nc + lax.axis_index("c")
    base = wid * _BPW
    pltpu.sync_copy(idx_hbm.at[pl.ds(wid * _NCHUNK, _NCHUNK)], idx_v)
    cps = [pltpu.async_copy(table_hbm.at[idx_v.at[j]],
                            rows_v.at[pl.ds(j * _CH, _CH)], sem)
           for j in range(_NCHUNK)]
    for cp in cps:
        cp.wait()
    pltpu.sync_copy(rows_v, out_hbm.at[pl.ds(base, _BPW)])


def _sc_gather(emb, idx2d):
    info = plsc.get_sparse_core_info()
    nc = info.num_cores
    mesh = plsc.VectorSubcoreMesh(core_axis_name="c", subcore_axis_name="s")
    fn = functools.partial(
        pl.kernel, mesh=mesh,
        out_type=jax.ShapeDtypeStruct((_N_TOK, _D), jnp.float32),
        scratch_types=[pltpu.VMEM((_NCHUNK, _CH), jnp.int32),
                       pltpu.VMEM((_BPW, _D), jnp.float32),
                       pltpu.SemaphoreType.DMA],
    )(functools.partial(_gather_body, nc))
    return fn(emb, idx2d)


def _fix_body(z_ref, q_ref, o_ref):
    o_ref[...] = z_ref[...] + (q_ref[...] - z_ref[...])


def _fix(z_flat, zq_flat):
    return pl.pallas_call(
        _fix_body,
        grid=(_GRID,),
        in_specs=[pl.BlockSpec((_TN, _D), lambda i: (i, 0)),
                  pl.BlockSpec((_TN, _D), lambda i: (i, 0))],
        out_specs=pl.BlockSpec((_TN, _D), lambda i: (i, 0)),
        out_shape=jax.ShapeDtypeStruct((_N_TOK, _D), jnp.float32),
    )(z_flat, zq_flat)


def kernel(z, emb_weight):
    z_flat = z.reshape(-1, _D)
    emb_t = emb_weight.T
    idx, loss_sum = _vq_argmin(z_flat, emb_t)
    zq_flat = _sc_gather(emb_weight, idx.reshape(_NW * _NCHUNK, _CH))
    zq_out = _fix(z_flat, zq_flat).reshape(z.shape)
    m = loss_sum / jnp.float32(z.size)
    loss = m + _COMMIT * m
    return (zq_out, loss)


# speedup vs baseline: 1.0467x; 1.0467x over previous
import functools

import jax
import jax.numpy as jnp
import numpy as np
from jax import lax
from jax.experimental import pallas as pl
from jax.experimental.pallas import tpu as pltpu
from jax.experimental.pallas import tpu_sc as plsc

_N_CODES = 8192
_D = 256
_N_TOK = 8192
_TN = 256
_GRID = _N_TOK // _TN

_COMMIT = 0.25


_WIN = 2048
_BIG = np.int32(1 << 30)


def _round_bf16(x):
    u = lax.bitcast_convert_type(x, jnp.uint32)
    r = (u + np.uint32(0x7FFF) + ((u >> 16) & np.uint32(1))) \
        & np.uint32(0xFFFF0000)
    return lax.bitcast_convert_type(r, jnp.float32)


def _argmin_body(prec, z_ref, emb_ref, idx_ref, loss_ref, esq_ref, embb_ref):
    i = pl.program_id(0)

    @pl.when(i == 0)
    def _init():
        et = emb_ref[...]
        esq_ref[...] = lax.dot_general(
            jnp.ones((1, _D), jnp.float32), et * et,
            (((1,), (1,)), ((), ())),
            precision=lax.Precision.HIGHEST,
            preferred_element_type=jnp.float32)
        embb_ref[...] = et.astype(jnp.bfloat16)
        loss_ref[...] = jnp.zeros_like(loss_ref)

    zt = z_ref[...]
    zsq = jnp.sum(zt * zt, axis=1, keepdims=True)
    zb2 = (zt + zt).astype(jnp.bfloat16)
    accv = acci = accf = None
    for w in range(_N_CODES // _WIN):
        mm2 = lax.dot_general(zb2, embb_ref[w * _WIN:(w + 1) * _WIN, :],
                              (((1,), (1,)), ((), ())),
                              preferred_element_type=jnp.float32)
        dw = (zsq + esq_ref[:, w * _WIN:(w + 1) * _WIN]) - mm2
        wmin = jnp.min(dw, axis=1, keepdims=True)
        ii = lax.broadcasted_iota(jnp.int32, dw.shape, 1) + np.int32(w * _WIN)
        warg = jnp.min(jnp.where(dw == wmin, ii, _BIG),
                       axis=1, keepdims=True)
        if w == 0:
            accv, acci, accf = _round_bf16(wmin), warg, wmin
        else:
            take_acc = (accv < wmin) | ((accv == wmin) & (acci < warg))
            accv = _round_bf16(jnp.where(take_acc, accv, wmin))
            acci = jnp.where(take_acc, acci, warg)
            accf = jnp.where(take_acc, accf, wmin)
    idx_ref[0] = acci
    loss_ref[...] = loss_ref[...] + jnp.sum(accf, axis=(0, 1), keepdims=True)


def _vq_argmin(z_flat, emb, prec=lax.Precision.DEFAULT):
    idx3, loss = pl.pallas_call(
        functools.partial(_argmin_body, prec),
        grid=(_GRID,),
        in_specs=[pl.BlockSpec((_TN, _D), lambda i: (i, 0)),
                  pl.BlockSpec((_N_CODES, _D), lambda i: (0, 0))],
        out_specs=[pl.BlockSpec((1, _TN, 1), lambda i: (i, 0, 0)),
                   pl.BlockSpec((1, 1), lambda i: (0, 0))],
        out_shape=[jax.ShapeDtypeStruct((_GRID, _TN, 1), jnp.int32),
                   jax.ShapeDtypeStruct((1, 1), jnp.float32)],
        scratch_shapes=[pltpu.VMEM((1, _N_CODES), jnp.float32),
                        pltpu.VMEM((_N_CODES, _D), jnp.bfloat16)],
    )(z_flat, emb)
    return idx3.reshape(-1), loss[0, 0]


_NW = 32
_BPW = _N_TOK // _NW
_CH = 128
_NCHUNK = _BPW // _CH


def _gather_body(nc, table_hbm, idx_hbm, out_hbm, idx_v, rows_v, sem):
    wid = lax.axis_index("s") * nc + lax.axis_index("c")
    base = wid * _BPW
    pltpu.sync_copy(idx_hbm.at[pl.ds(wid * _NCHUNK, _NCHUNK)], idx_v)
    cps = [pltpu.async_copy(table_hbm.at[idx_v.at[j]],
                            rows_v.at[pl.ds(j * _CH, _CH)], sem)
           for j in range(_NCHUNK)]
    for cp in cps:
        cp.wait()
    pltpu.sync_copy(rows_v, out_hbm.at[pl.ds(base, _BPW)])


def _sc_gather(emb, idx2d):
    info = plsc.get_sparse_core_info()
    nc = info.num_cores
    mesh = plsc.VectorSubcoreMesh(core_axis_name="c", subcore_axis_name="s")
    fn = functools.partial(
        pl.kernel, mesh=mesh,
        out_type=jax.ShapeDtypeStruct((_N_TOK, _D), jnp.float32),
        scratch_types=[pltpu.VMEM((_NCHUNK, _CH), jnp.int32),
                       pltpu.VMEM((_BPW, _D), jnp.float32),
                       pltpu.SemaphoreType.DMA],
    )(functools.partial(_gather_body, nc))
    return fn(emb, idx2d)


def _fix_body(z_ref, q_ref, o_ref):
    o_ref[...] = z_ref[...] + (q_ref[...] - z_ref[...])


def _fix(z_flat, zq_flat):
    return pl.pallas_call(
        _fix_body,
        grid=(_GRID,),
        in_specs=[pl.BlockSpec((_TN, _D), lambda i: (i, 0)),
                  pl.BlockSpec((_TN, _D), lambda i: (i, 0))],
        out_specs=pl.BlockSpec((_TN, _D), lambda i: (i, 0)),
        out_shape=jax.ShapeDtypeStruct((_N_TOK, _D), jnp.float32),
    )(z_flat, zq_flat)


def kernel(z, emb_weight):
    z_flat = z.reshape(-1, _D)
    idx, loss_sum = _vq_argmin(z_flat, emb_weight)
    zq_flat = _sc_gather(emb_weight, idx.reshape(_NW * _NCHUNK, _CH))
    zq_out = _fix(z_flat, zq_flat).reshape(z.shape)
    m = loss_sum / jnp.float32(z.size)
    loss = m + _COMMIT * m
    return (zq_out, loss)


# speedup vs baseline: 1.1948x; 1.1415x over previous
import functools

import jax
import jax.numpy as jnp
import numpy as np
from jax import lax
from jax.experimental import pallas as pl
from jax.experimental.pallas import tpu as pltpu
from jax.experimental.pallas import tpu_sc as plsc

_N_CODES = 8192
_D = 256
_N_TOK = 8192
_TN = 256
_GRID = _N_TOK // _TN

_COMMIT = 0.25


_WIN = 2048
_BIG = np.int32(1 << 30)


def _round_bf16(x):
    u = lax.bitcast_convert_type(x, jnp.uint32)
    r = (u + np.uint32(0x7FFF) + ((u >> 16) & np.uint32(1))) \
        & np.uint32(0xFFFF0000)
    return lax.bitcast_convert_type(r, jnp.float32)


def _argmin_body(prec, z_ref, emb_ref, idx_ref, loss_ref, esq_ref, embb_ref):
    i = pl.program_id(0)

    @pl.when(i == 0)
    def _init():
        et = emb_ref[...]
        esq_ref[...] = lax.dot_general(
            jnp.ones((1, _D), jnp.float32), et * et,
            (((1,), (1,)), ((), ())),
            precision=lax.Precision.HIGHEST,
            preferred_element_type=jnp.float32)
        embb_ref[...] = et.astype(jnp.bfloat16)
        loss_ref[...] = jnp.zeros_like(loss_ref)

    zt = z_ref[...]
    zsq = jnp.sum(zt * zt, axis=1, keepdims=True)
    zb2 = (zt + zt).astype(jnp.bfloat16)
    accv = acci = accf = None
    for w in range(_N_CODES // _WIN):
        mm2 = lax.dot_general(zb2, embb_ref[w * _WIN:(w + 1) * _WIN, :],
                              (((1,), (1,)), ((), ())),
                              preferred_element_type=jnp.float32)
        dw = (zsq + esq_ref[:, w * _WIN:(w + 1) * _WIN]) - mm2
        wmin = jnp.min(dw, axis=1, keepdims=True)
        ii = lax.broadcasted_iota(jnp.int32, dw.shape, 1) + np.int32(w * _WIN)
        warg = jnp.min(jnp.where(dw == wmin, ii, _BIG),
                       axis=1, keepdims=True)
        if w == 0:
            accv, acci, accf = _round_bf16(wmin), warg, wmin
        else:
            take_acc = (accv < wmin) | ((accv == wmin) & (acci < warg))
            accv = _round_bf16(jnp.where(take_acc, accv, wmin))
            acci = jnp.where(take_acc, acci, warg)
            accf = jnp.where(take_acc, accf, wmin)
    idx_ref[0] = acci
    loss_ref[...] = loss_ref[...] + jnp.sum(accf, axis=(0, 1), keepdims=True)


def _vq_argmin(z_flat, emb, prec=lax.Precision.DEFAULT):
    idx3, loss = pl.pallas_call(
        functools.partial(_argmin_body, prec),
        grid=(_GRID,),
        in_specs=[pl.BlockSpec((_TN, _D), lambda i: (i, 0)),
                  pl.BlockSpec((_N_CODES, _D), lambda i: (0, 0))],
        out_specs=[pl.BlockSpec((1, _TN, 1), lambda i: (i, 0, 0)),
                   pl.BlockSpec((1, 1), lambda i: (0, 0))],
        out_shape=[jax.ShapeDtypeStruct((_GRID, _TN, 1), jnp.int32),
                   jax.ShapeDtypeStruct((1, 1), jnp.float32)],
        scratch_shapes=[pltpu.VMEM((1, _N_CODES), jnp.float32),
                        pltpu.VMEM((_N_CODES, _D), jnp.bfloat16)],
    )(z_flat, emb)
    return idx3.reshape(-1), loss[0, 0]


_NW = 32
_BPW = _N_TOK // _NW
_CH = 128
_NCHUNK = _BPW // _CH


def _gather_body(nc, table_hbm, idx_hbm, out_hbm, idx_v, rows_v, sem):
    wid = lax.axis_index("s") * nc + lax.axis_index("c")
    base = wid * _BPW
    pltpu.sync_copy(idx_hbm.at[pl.ds(wid * _NCHUNK, _NCHUNK)], idx_v)
    cps = [pltpu.async_copy(table_hbm.at[idx_v.at[j]],
                            rows_v.at[pl.ds(j * _CH, _CH)], sem)
           for j in range(_NCHUNK)]
    for cp in cps:
        cp.wait()
    pltpu.sync_copy(rows_v, out_hbm.at[pl.ds(base, _BPW)])


def _sc_gather(emb, idx2d):
    info = plsc.get_sparse_core_info()
    nc = info.num_cores
    mesh = plsc.VectorSubcoreMesh(core_axis_name="c", subcore_axis_name="s")
    fn = functools.partial(
        pl.kernel, mesh=mesh,
        out_type=jax.ShapeDtypeStruct((_N_TOK, _D), jnp.float32),
        scratch_types=[pltpu.VMEM((_NCHUNK, _CH), jnp.int32),
                       pltpu.VMEM((_BPW, _D), jnp.float32),
                       pltpu.SemaphoreType.DMA],
    )(functools.partial(_gather_body, nc))
    return fn(emb, idx2d)


def kernel(z, emb_weight):
    z_flat = z.reshape(-1, _D)
    idx, loss_sum = _vq_argmin(z_flat, emb_weight)
    zq_flat = _sc_gather(emb_weight, idx.reshape(_NW * _NCHUNK, _CH))
    zq_out = zq_flat.reshape(z.shape)
    m = loss_sum / jnp.float32(z.size)
    loss = m + _COMMIT * m
    return (zq_out, loss)


# speedup vs baseline: 1.3763x; 1.1520x over previous
import functools

import jax
import jax.numpy as jnp
import numpy as np
from jax import lax
from jax.experimental import pallas as pl
from jax.experimental.pallas import tpu as pltpu
from jax.experimental.pallas import tpu_sc as plsc

_N_CODES = 8192
_D = 256
_N_TOK = 8192
_TN = 256
_GRID = _N_TOK // _TN

_COMMIT = 0.25


_WIN = 2048
_BIGF = np.float32(1e9)


def _round_bf16(x):
    u = lax.bitcast_convert_type(x, jnp.uint32)
    r = (u + np.uint32(0x7FFF) + ((u >> 16) & np.uint32(1))) \
        & np.uint32(0xFFFF0000)
    return lax.bitcast_convert_type(r, jnp.float32)


def _argmin_body(prec, z_ref, emb_ref, idx_ref, loss_ref, esq_ref, embb_ref):
    i = pl.program_id(0)

    @pl.when(i == 0)
    def _init():
        et = emb_ref[...]
        esq_ref[...] = lax.dot_general(
            jnp.ones((1, _D), jnp.float32), et * et,
            (((1,), (1,)), ((), ())),
            precision=lax.Precision.HIGHEST,
            preferred_element_type=jnp.float32)
        embb_ref[...] = et.astype(jnp.bfloat16)
        loss_ref[...] = jnp.zeros_like(loss_ref)

    zt = z_ref[...]
    zsq = jnp.sum(zt * zt, axis=1, keepdims=True)
    zb2 = (zt + zt).astype(jnp.bfloat16)
    accv = acci = accf = None
    for w in range(_N_CODES // _WIN):
        mm2 = lax.dot_general(zb2, embb_ref[w * _WIN:(w + 1) * _WIN, :],
                              (((1,), (1,)), ((), ())),
                              preferred_element_type=jnp.float32)
        dw = (zsq + esq_ref[:, w * _WIN:(w + 1) * _WIN]) - mm2
        wmin = jnp.min(dw, axis=1, keepdims=True)
        ii = (lax.broadcasted_iota(jnp.int32, dw.shape, 1)
              .astype(jnp.float32) + np.float32(w * _WIN))
        warg = jnp.min(jnp.where(dw == wmin, ii, _BIGF),
                       axis=1, keepdims=True)
        if w == 0:
            accv, acci, accf = _round_bf16(wmin), warg, wmin
        else:
            take_acc = (accv < wmin) | ((accv == wmin) & (acci < warg))
            accv = _round_bf16(jnp.where(take_acc, accv, wmin))
            acci = jnp.where(take_acc, acci, warg)
            accf = jnp.where(take_acc, accf, wmin)
    idx_ref[0] = acci.astype(jnp.int32)
    loss_ref[...] = loss_ref[...] + jnp.sum(accf, axis=(0, 1), keepdims=True)


def _vq_argmin(z_flat, emb, prec=lax.Precision.DEFAULT):
    idx3, loss = pl.pallas_call(
        functools.partial(_argmin_body, prec),
        grid=(_GRID,),
        in_specs=[pl.BlockSpec((_TN, _D), lambda i: (i, 0)),
                  pl.BlockSpec((_N_CODES, _D), lambda i: (0, 0))],
        out_specs=[pl.BlockSpec((1, _TN, 1), lambda i: (i, 0, 0)),
                   pl.BlockSpec((1, 1), lambda i: (0, 0))],
        out_shape=[jax.ShapeDtypeStruct((_GRID, _TN, 1), jnp.int32),
                   jax.ShapeDtypeStruct((1, 1), jnp.float32)],
        scratch_shapes=[pltpu.VMEM((1, _N_CODES), jnp.float32),
                        pltpu.VMEM((_N_CODES, _D), jnp.bfloat16)],
    )(z_flat, emb)
    return idx3.reshape(-1), loss[0, 0]


_NW = 32
_BPW = _N_TOK // _NW
_CH = 128
_NCHUNK = _BPW // _CH


def _gather_body(nc, table_hbm, idx_hbm, out_hbm, idx_v, rows_v, sem):
    wid = lax.axis_index("s") * nc + lax.axis_index("c")
    base = wid * _BPW
    pltpu.sync_copy(idx_hbm.at[pl.ds(wid * _NCHUNK, _NCHUNK)], idx_v)
    cps = [pltpu.async_copy(table_hbm.at[idx_v.at[j]],
                            rows_v.at[pl.ds(j * _CH, _CH)], sem)
           for j in range(_NCHUNK)]
    for cp in cps:
        cp.wait()
    pltpu.sync_copy(rows_v, out_hbm.at[pl.ds(base, _BPW)])


def _sc_gather(emb, idx2d):
    info = plsc.get_sparse_core_info()
    nc = info.num_cores
    mesh = plsc.VectorSubcoreMesh(core_axis_name="c", subcore_axis_name="s")
    fn = functools.partial(
        pl.kernel, mesh=mesh,
        out_type=jax.ShapeDtypeStruct((_N_TOK, _D), jnp.float32),
        scratch_types=[pltpu.VMEM((_NCHUNK, _CH), jnp.int32),
                       pltpu.VMEM((_BPW, _D), jnp.float32),
                       pltpu.SemaphoreType.DMA],
    )(functools.partial(_gather_body, nc))
    return fn(emb, idx2d)


def kernel(z, emb_weight):
    z_flat = z.reshape(-1, _D)
    idx, loss_sum = _vq_argmin(z_flat, emb_weight)
    zq_flat = _sc_gather(emb_weight, idx.reshape(_NW * _NCHUNK, _CH))
    zq_out = zq_flat.reshape(z.shape)
    m = loss_sum / jnp.float32(z.size)
    loss = m + _COMMIT * m
    return (zq_out, loss)
